# Initial kernel scaffold; baseline (speedup 1.0000x reference)
import jax
import jax.numpy as jnp
from jax.experimental import pallas as pl


def kernel(x, edge_index, W1l, W1r, b1, W2l, W2r, b2):
    raise NotImplementedError("write your pallas kernel here")


# speedup vs baseline: 6.0859x; 6.0859x over previous
import jax
import jax.numpy as jnp
from jax import lax
from jax.experimental import pallas as pl
from jax.experimental.pallas import tpu as pltpu
from jax.experimental.pallas import tpu_sc as plsc

NN = 10000
NP = 10240
FF = 128
FH = 64
EE = 320000
EP = 327680
EPT = EP // 16
CHUNK = 512
NCHUNK = EPT // CHUNK
RPT = NP // 16


def _tc_proj_body(x_ref, wl_ref, wr_ref, b_ref, pa_ref, pb_ref, q_ref):
    xb = x_ref[...]
    p = jnp.dot(xb, wl_ref[...], preferred_element_type=jnp.float32)
    pa_ref[...] = p[:, :FH]
    pb_ref[...] = p[:, FH:]
    q_ref[...] = (
        jnp.dot(xb, wr_ref[...], preferred_element_type=jnp.float32)
        + b_ref[...]
    )


def _tc_proj(xp, wl, wr, b):
    blk = 1024
    grid = NP // blk
    hspec = pl.BlockSpec((blk, FH), lambda i: (i, 0))
    return pl.pallas_call(
        _tc_proj_body,
        grid=(grid,),
        in_specs=[
            pl.BlockSpec((blk, FF), lambda i: (i, 0)),
            pl.BlockSpec((FF, FF), lambda i: (0, 0)),
            pl.BlockSpec((FF, FF), lambda i: (0, 0)),
            pl.BlockSpec((1, FF), lambda i: (0, 0)),
        ],
        out_specs=[hspec, hspec, pl.BlockSpec((blk, FF), lambda i: (i, 0))],
        out_shape=[
            jax.ShapeDtypeStruct((NP, FH), jnp.float32),
            jax.ShapeDtypeStruct((NP, FH), jnp.float32),
            jax.ShapeDtypeStruct((NP, FF), jnp.float32),
        ],
    )(xp, wl, wr, b.reshape(1, FF))


def _tc_comb_proj_body(sa_ref, sb_ref, pa_ref, pb_ref, q_ref, d_ref,
                       wl_ref, wr_ref, b_ref,
                       p2a_ref, p2b_ref, q2_ref, inv_ref):
    inv = 1.0 / (1.0 + d_ref[...][:, :1])
    inv_ref[...] = inv
    s = jnp.concatenate([sa_ref[...], sb_ref[...]], axis=1)
    p = jnp.concatenate([pa_ref[...], pb_ref[...]], axis=1)
    h = jnp.maximum((s + p) * inv + q_ref[...], 0.0)
    p2 = jnp.dot(h, wl_ref[...], preferred_element_type=jnp.float32)
    p2a_ref[...] = p2[:, :FH]
    p2b_ref[...] = p2[:, FH:]
    q2_ref[...] = (
        jnp.dot(h, wr_ref[...], preferred_element_type=jnp.float32)
        + b_ref[...]
    )


def _tc_comb_proj(sa, sb, pa, pb, q, deg, wl, wr, b):
    blk = 1024
    grid = NP // blk
    rspec = pl.BlockSpec((blk, FF), lambda i: (i, 0))
    hspec = pl.BlockSpec((blk, FH), lambda i: (i, 0))
    dspec = pl.BlockSpec((blk, 16), lambda i: (i, 0))
    wspec = pl.BlockSpec((FF, FF), lambda i: (0, 0))
    return pl.pallas_call(
        _tc_comb_proj_body,
        grid=(grid,),
        in_specs=[hspec, hspec, hspec, hspec, rspec, dspec, wspec, wspec,
                  pl.BlockSpec((1, FF), lambda i: (0, 0))],
        out_specs=[hspec, hspec, rspec,
                   pl.BlockSpec((blk, 1), lambda i: (i, 0))],
        out_shape=[
            jax.ShapeDtypeStruct((NP, FH), jnp.float32),
            jax.ShapeDtypeStruct((NP, FH), jnp.float32),
            jax.ShapeDtypeStruct((NP, FF), jnp.float32),
            jax.ShapeDtypeStruct((NP, 1), jnp.float32),
        ],
    )(sa, sb, pa, pb, q, deg, wl, wr, b.reshape(1, FF))


def _tc_comb_body(sa_ref, sb_ref, pa_ref, pb_ref, q_ref, inv_ref, o_ref):
    s = jnp.concatenate([sa_ref[...], sb_ref[...]], axis=1)
    p = jnp.concatenate([pa_ref[...], pb_ref[...]], axis=1)
    o_ref[...] = jnp.maximum((s + p) * inv_ref[...] + q_ref[...], 0.0)


def _tc_comb(sa, sb, pa, pb, q, inv):
    blk = 1024
    grid = NP // blk
    rspec = pl.BlockSpec((blk, FF), lambda i: (i, 0))
    hspec = pl.BlockSpec((blk, FH), lambda i: (i, 0))
    ispec = pl.BlockSpec((blk, 1), lambda i: (i, 0))
    return pl.pallas_call(
        _tc_comb_body,
        grid=(grid,),
        in_specs=[hspec, hspec, hspec, hspec, rspec, ispec],
        out_specs=rspec,
        out_shape=jax.ShapeDtypeStruct((NP, FF), jnp.float32),
    )(sa, sb, pa, pb, q, inv)


def _make_sc_scatter(compute_deg: bool):
    mesh = plsc.VectorSubcoreMesh(core_axis_name="c", subcore_axis_name="s")
    out_type = [jax.ShapeDtypeStruct((2, NP, FH), jnp.float32)]
    if compute_deg:
        out_type.append(jax.ShapeDtypeStruct((NP, 16), jnp.float32))
    scratch = [
        pltpu.VMEM((4, 128), jnp.int32),
        pltpu.VMEM((4, 128), jnp.int32),
        pltpu.VMEM((CHUNK, FH), jnp.float32),
        pltpu.VMEM((16, FH), jnp.float32),
        pltpu.VMEM_SHARED((NP, FH), jnp.float32),
        pltpu.SemaphoreType.DMA,
    ]
    if compute_deg:
        scratch += [
            pltpu.VMEM((128, 16), jnp.float32),
            pltpu.VMEM((64, 16), jnp.float32),
            pltpu.VMEM_SHARED((NP, 16), jnp.float32),
        ]

    def body(pa_hbm, pb_hbm, src_hbm, dst_hbm, *refs):
        if compute_deg:
            (sacc_hbm, deg_hbm, srcv, dstv, rows, zbuf, acc, sem,
             onesb, zb16, dacc) = refs
        else:
            (sacc_hbm, srcv, dstv, rows, zbuf, acc, sem) = refs
        c = lax.axis_index("c")
        s = lax.axis_index("s")

        z16 = jnp.zeros((16,), jnp.float32)
        for r in range(16):
            for k in range(FH // 16):
                zbuf[r, pl.ds(k * 16, 16)] = z16

        def zero_acc(i, _):
            pltpu.sync_copy(zbuf, acc.at[pl.ds(s * RPT + i * 16, 16)])
            return 0
        lax.fori_loop(0, RPT // 16, zero_acc, 0)

        if compute_deg:
            o16 = jnp.ones((16,), jnp.float32)

            def fill_ones(i, _):
                onesb[i, pl.ds(0, 16)] = o16
                return 0
            lax.fori_loop(0, 128, fill_ones, 0)

            def zero_zb16(i, _):
                zb16[i, pl.ds(0, 16)] = z16
                return 0
            lax.fori_loop(0, 64, zero_zb16, 0)

            @pl.when(c == 0)
            def _():
                def zero_dacc(i, _):
                    pltpu.sync_copy(
                        zb16, dacc.at[pl.ds(s * RPT + i * 64, 64)])
                    return 0
                lax.fori_loop(0, RPT // 64, zero_dacc, 0)

        plsc.subcore_barrier()

        def run(p_hbm, do_deg):
            def step(g, _):
                row0 = s * (EPT // 128) + g * 4
                pltpu.sync_copy(src_hbm.at[pl.ds(row0, 4)], srcv)
                pltpu.sync_copy(dst_hbm.at[pl.ds(row0, 4)], dstv)
                cps = [
                    pltpu.async_copy(
                        p_hbm.at[srcv.at[j]],
                        rows.at[pl.ds(j * 128, 128)], sem)
                    for j in range(4)
                ]
                for cp in cps:
                    cp.wait()
                for j in range(4):
                    pltpu.sync_copy(
                        rows.at[pl.ds(j * 128, 128)], acc.at[dstv.at[j]],
                        add=True)
                if do_deg:
                    for j in range(4):
                        pltpu.sync_copy(
                            onesb, dacc.at[dstv.at[j]], add=True)
                return 0
            lax.fori_loop(0, NCHUNK, step, 0)

        @pl.when(c == 0)
        def _():
            run(pa_hbm, compute_deg)

        @pl.when(c == 1)
        def _():
            run(pb_hbm, False)

        plsc.subcore_barrier()

        pltpu.sync_copy(
            acc.at[pl.ds(s * RPT, RPT)],
            sacc_hbm.at[c, pl.ds(s * RPT, RPT)])

        if compute_deg:
            @pl.when(c == 0)
            def _():
                pltpu.sync_copy(
                    dacc.at[pl.ds(s * RPT, RPT)],
                    deg_hbm.at[pl.ds(s * RPT, RPT)])

    return pl.kernel(
        body, mesh=mesh, out_type=out_type, scratch_types=scratch,
        compiler_params=pltpu.CompilerParams(use_tc_tiling_on_sc=False))


_sc_scatter_deg = _make_sc_scatter(True)
_sc_scatter = _make_sc_scatter(False)


@jax.jit
def kernel(x, edge_index, W1l, W1r, b1, W2l, W2r, b2):
    xp = jnp.zeros((NP, FF), jnp.float32).at[:NN].set(x)
    pad = EP - EE
    srcp = jnp.concatenate(
        [edge_index[0], jnp.zeros((pad,), jnp.int32)]).reshape(EP // 128, 128)
    dstp = jnp.concatenate(
        [edge_index[1], jnp.full((pad,), NN, jnp.int32)]).reshape(EP // 128, 128)

    p1a, p1b, q1 = _tc_proj(xp, W1l, W1r, b1)
    sacc1, deg = _sc_scatter_deg(p1a, p1b, srcp, dstp)
    p2a, p2b, q2, inv = _tc_comb_proj(sacc1[0], sacc1[1], p1a, p1b, q1,
                                      deg, W2l, W2r, b2)
    (sacc2,) = _sc_scatter(p2a, p2b, srcp, dstp)
    out = _tc_comb(sacc2[0], sacc2[1], p2a, p2b, q2, inv)
    return out[:NN]


# speedup vs baseline: 7.0653x; 1.1609x over previous
import jax
import jax.numpy as jnp
from jax import lax
from jax.experimental import pallas as pl
from jax.experimental.pallas import tpu as pltpu
from jax.experimental.pallas import tpu_sc as plsc

NN = 10000
NP = 10240
FF = 128
FH = 64
EE = 320000
EP = 327680
EPT = EP // 16
CHUNK = 512
NCHUNK = EPT // CHUNK
RPT = NP // 16


def _tc_proj_body(x_ref, wl_ref, wr_ref, b_ref, pa_ref, pb_ref, q_ref):
    xb = x_ref[...]
    p = jnp.dot(xb, wl_ref[...], preferred_element_type=jnp.float32)
    pa_ref[...] = p[:, :FH]
    pb_ref[...] = p[:, FH:]
    q_ref[...] = (
        jnp.dot(xb, wr_ref[...], preferred_element_type=jnp.float32)
        + b_ref[...]
    )


def _tc_proj(xp, wl, wr, b):
    blk = 1024
    grid = NP // blk
    hspec = pl.BlockSpec((blk, FH), lambda i: (i, 0))
    return pl.pallas_call(
        _tc_proj_body,
        grid=(grid,),
        in_specs=[
            pl.BlockSpec((blk, FF), lambda i: (i, 0)),
            pl.BlockSpec((FF, FF), lambda i: (0, 0)),
            pl.BlockSpec((FF, FF), lambda i: (0, 0)),
            pl.BlockSpec((1, FF), lambda i: (0, 0)),
        ],
        out_specs=[hspec, hspec, pl.BlockSpec((blk, FF), lambda i: (i, 0))],
        out_shape=[
            jax.ShapeDtypeStruct((NP, FH), jnp.float32),
            jax.ShapeDtypeStruct((NP, FH), jnp.float32),
            jax.ShapeDtypeStruct((NP, FF), jnp.float32),
        ],
    )(xp, wl, wr, b.reshape(1, FF))


def _tc_comb_proj_body(sa_ref, sb_ref, pa_ref, pb_ref, q_ref, d_ref,
                       wl_ref, wr_ref, b_ref,
                       p2a_ref, p2b_ref, q2_ref, inv_ref):
    inv = 1.0 / (1.0 + d_ref[...][:, :1])
    inv_ref[...] = inv
    s = jnp.concatenate([sa_ref[...], sb_ref[...]], axis=1)
    p = jnp.concatenate([pa_ref[...], pb_ref[...]], axis=1)
    h = jnp.maximum((s + p) * inv + q_ref[...], 0.0)
    p2 = jnp.dot(h, wl_ref[...], preferred_element_type=jnp.float32)
    p2a_ref[...] = p2[:, :FH]
    p2b_ref[...] = p2[:, FH:]
    q2_ref[...] = (
        jnp.dot(h, wr_ref[...], preferred_element_type=jnp.float32)
        + b_ref[...]
    )


def _tc_comb_proj(sa, sb, pa, pb, q, deg, wl, wr, b):
    blk = 1024
    grid = NP // blk
    rspec = pl.BlockSpec((blk, FF), lambda i: (i, 0))
    hspec = pl.BlockSpec((blk, FH), lambda i: (i, 0))
    dspec = pl.BlockSpec((blk, 16), lambda i: (i, 0))
    wspec = pl.BlockSpec((FF, FF), lambda i: (0, 0))
    return pl.pallas_call(
        _tc_comb_proj_body,
        grid=(grid,),
        in_specs=[hspec, hspec, hspec, hspec, rspec, dspec, wspec, wspec,
                  pl.BlockSpec((1, FF), lambda i: (0, 0))],
        out_specs=[hspec, hspec, rspec,
                   pl.BlockSpec((blk, 1), lambda i: (i, 0))],
        out_shape=[
            jax.ShapeDtypeStruct((NP, FH), jnp.float32),
            jax.ShapeDtypeStruct((NP, FH), jnp.float32),
            jax.ShapeDtypeStruct((NP, FF), jnp.float32),
            jax.ShapeDtypeStruct((NP, 1), jnp.float32),
        ],
    )(sa, sb, pa, pb, q, deg, wl, wr, b.reshape(1, FF))


def _tc_comb_body(sa_ref, sb_ref, pa_ref, pb_ref, q_ref, inv_ref, o_ref):
    s = jnp.concatenate([sa_ref[...], sb_ref[...]], axis=1)
    p = jnp.concatenate([pa_ref[...], pb_ref[...]], axis=1)
    o_ref[...] = jnp.maximum((s + p) * inv_ref[...] + q_ref[...], 0.0)


def _tc_comb(sa, sb, pa, pb, q, inv):
    blk = 1024
    grid = NP // blk
    rspec = pl.BlockSpec((blk, FF), lambda i: (i, 0))
    hspec = pl.BlockSpec((blk, FH), lambda i: (i, 0))
    ispec = pl.BlockSpec((blk, 1), lambda i: (i, 0))
    return pl.pallas_call(
        _tc_comb_body,
        grid=(grid,),
        in_specs=[hspec, hspec, hspec, hspec, rspec, ispec],
        out_specs=rspec,
        out_shape=jax.ShapeDtypeStruct((NP, FF), jnp.float32),
    )(sa, sb, pa, pb, q, inv)


def _make_sc_scatter(compute_deg: bool):
    mesh = plsc.VectorSubcoreMesh(core_axis_name="c", subcore_axis_name="s")
    out_type = [jax.ShapeDtypeStruct((2, NP, FH), jnp.float32)]
    if compute_deg:
        out_type.append(jax.ShapeDtypeStruct((NP, 16), jnp.float32))
    scratch = [
        pltpu.VMEM((4, 128), jnp.int32),
        pltpu.VMEM((4, 128), jnp.int32),
        pltpu.VMEM((CHUNK, FH), jnp.float32),
        pltpu.VMEM((4, 128), jnp.int32),
        pltpu.VMEM((4, 128), jnp.int32),
        pltpu.VMEM((CHUNK, FH), jnp.float32),
        pltpu.VMEM((16, FH), jnp.float32),
        pltpu.VMEM_SHARED((NP, FH), jnp.float32),
        pltpu.SemaphoreType.DMA,
        pltpu.SemaphoreType.DMA,
        pltpu.SemaphoreType.DMA,
        pltpu.SemaphoreType.DMA,
    ]
    if compute_deg:
        scratch += [
            pltpu.VMEM((128, 16), jnp.float32),
            pltpu.VMEM((64, 16), jnp.float32),
            pltpu.VMEM_SHARED((NP, 16), jnp.float32),
        ]

    def body(pa_hbm, pb_hbm, src_hbm, dst_hbm, *refs):
        if compute_deg:
            (sacc_hbm, deg_hbm, srcv0, dstv0, rows0, srcv1, dstv1, rows1,
             zbuf, acc, sg0, sg1, ss0, ss1, onesb, zb16, dacc) = refs
        else:
            (sacc_hbm, srcv0, dstv0, rows0, srcv1, dstv1, rows1,
             zbuf, acc, sg0, sg1, ss0, ss1) = refs
        buf0 = (srcv0, dstv0, rows0, sg0, ss0)
        buf1 = (srcv1, dstv1, rows1, sg1, ss1)
        c = lax.axis_index("c")
        s = lax.axis_index("s")

        z16 = jnp.zeros((16,), jnp.float32)
        for r in range(16):
            for k in range(FH // 16):
                zbuf[r, pl.ds(k * 16, 16)] = z16

        def zero_acc(i, _):
            pltpu.sync_copy(zbuf, acc.at[pl.ds(s * RPT + i * 16, 16)])
            return 0
        lax.fori_loop(0, RPT // 16, zero_acc, 0)

        if compute_deg:
            o16 = jnp.ones((16,), jnp.float32)

            def fill_ones(i, _):
                onesb[i, pl.ds(0, 16)] = o16
                return 0
            lax.fori_loop(0, 128, fill_ones, 0)

            def zero_zb16(i, _):
                zb16[i, pl.ds(0, 16)] = z16
                return 0
            lax.fori_loop(0, 64, zero_zb16, 0)

            @pl.when(c == 0)
            def _():
                def zero_dacc(i, _):
                    pltpu.sync_copy(
                        zb16, dacc.at[pl.ds(s * RPT + i * 64, 64)])
                    return 0
                lax.fori_loop(0, RPT // 64, zero_dacc, 0)

        plsc.subcore_barrier()

        def run(p_hbm, do_deg):
            def load_idx(buf, g):
                row0 = s * (EPT // 128) + g * 4
                pltpu.sync_copy(src_hbm.at[pl.ds(row0, 4)], buf[0])
                pltpu.sync_copy(dst_hbm.at[pl.ds(row0, 4)], buf[1])

            def issue_g(buf):
                for j in range(4):
                    pltpu.async_copy(
                        p_hbm.at[buf[0].at[j]],
                        buf[2].at[pl.ds(j * 128, 128)], buf[3])

            def wait_g(buf):
                for j in range(4):
                    pltpu.make_async_copy(
                        p_hbm.at[buf[0].at[j]],
                        buf[2].at[pl.ds(j * 128, 128)], buf[3]).wait()

            def issue_s(buf):
                for j in range(4):
                    pltpu.async_copy(
                        buf[2].at[pl.ds(j * 128, 128)],
                        acc.at[buf[1].at[j]], buf[4], add=True)
                if do_deg:
                    for j in range(4):
                        pltpu.async_copy(
                            onesb, dacc.at[buf[1].at[j]], buf[4], add=True)

            def wait_s(buf):
                for j in range(4):
                    pltpu.make_async_copy(
                        buf[2].at[pl.ds(j * 128, 128)],
                        acc.at[buf[1].at[j]], buf[4]).wait()
                if do_deg:
                    for j in range(4):
                        pltpu.make_async_copy(
                            onesb, dacc.at[buf[1].at[j]], buf[4]).wait()

            load_idx(buf0, 0)
            issue_g(buf0)
            wait_g(buf0)
            issue_s(buf0)
            load_idx(buf1, 1)
            issue_g(buf1)

            def step(u, _):
                wait_g(buf1)
                issue_s(buf1)
                wait_s(buf0)
                load_idx(buf0, 2 * u + 2)
                issue_g(buf0)
                wait_g(buf0)
                issue_s(buf0)
                wait_s(buf1)
                load_idx(buf1, 2 * u + 3)
                issue_g(buf1)
                return 0
            lax.fori_loop(0, (NCHUNK - 2) // 2, step, 0)

            wait_g(buf1)
            issue_s(buf1)
            wait_s(buf0)
            wait_s(buf1)

        @pl.when(c == 0)
        def _():
            run(pa_hbm, compute_deg)

        @pl.when(c == 1)
        def _():
            run(pb_hbm, False)

        plsc.subcore_barrier()

        pltpu.sync_copy(
            acc.at[pl.ds(s * RPT, RPT)],
            sacc_hbm.at[c, pl.ds(s * RPT, RPT)])

        if compute_deg:
            @pl.when(c == 0)
            def _():
                pltpu.sync_copy(
                    dacc.at[pl.ds(s * RPT, RPT)],
                    deg_hbm.at[pl.ds(s * RPT, RPT)])

    return pl.kernel(
        body, mesh=mesh, out_type=out_type, scratch_types=scratch,
        compiler_params=pltpu.CompilerParams(use_tc_tiling_on_sc=False))


_sc_scatter_deg = _make_sc_scatter(True)
_sc_scatter = _make_sc_scatter(False)


@jax.jit
def kernel(x, edge_index, W1l, W1r, b1, W2l, W2r, b2):
    xp = jnp.zeros((NP, FF), jnp.float32).at[:NN].set(x)
    pad = EP - EE
    srcp = jnp.concatenate(
        [edge_index[0], jnp.zeros((pad,), jnp.int32)]).reshape(EP // 128, 128)
    dstp = jnp.concatenate(
        [edge_index[1], jnp.full((pad,), NN, jnp.int32)]).reshape(EP // 128, 128)

    p1a, p1b, q1 = _tc_proj(xp, W1l, W1r, b1)
    sacc1, deg = _sc_scatter_deg(p1a, p1b, srcp, dstp)
    p2a, p2b, q2, inv = _tc_comb_proj(sacc1[0], sacc1[1], p1a, p1b, q1,
                                      deg, W2l, W2r, b2)
    (sacc2,) = _sc_scatter(p2a, p2b, srcp, dstp)
    out = _tc_comb(sacc2[0], sacc2[1], p2a, p2b, q2, inv)
    return out[:NN]


# speedup vs baseline: 7.5230x; 1.0648x over previous
import jax
import jax.numpy as jnp
from jax import lax
from jax.experimental import pallas as pl
from jax.experimental.pallas import tpu as pltpu
from jax.experimental.pallas import tpu_sc as plsc

NN = 10000
NP = 10240
FF = 128
FH = 64
EE = 320000
EP = 327680
EPT = EP // 16
CHUNK = 256
KSUB = CHUNK // 128
NBUF = 4
NCHUNK = EPT // CHUNK
RPT = NP // 16


def _tc_proj_body(x_ref, wl_ref, wr_ref, b_ref, pa_ref, pb_ref, q_ref):
    xb = x_ref[...]
    p = jnp.dot(xb, wl_ref[...], preferred_element_type=jnp.float32)
    pa_ref[...] = p[:, :FH]
    pb_ref[...] = p[:, FH:]
    q_ref[...] = (
        jnp.dot(xb, wr_ref[...], preferred_element_type=jnp.float32)
        + b_ref[...]
    )


def _tc_proj(xp, wl, wr, b):
    blk = 1024
    grid = NP // blk
    hspec = pl.BlockSpec((blk, FH), lambda i: (i, 0))
    return pl.pallas_call(
        _tc_proj_body,
        grid=(grid,),
        in_specs=[
            pl.BlockSpec((blk, FF), lambda i: (i, 0)),
            pl.BlockSpec((FF, FF), lambda i: (0, 0)),
            pl.BlockSpec((FF, FF), lambda i: (0, 0)),
            pl.BlockSpec((1, FF), lambda i: (0, 0)),
        ],
        out_specs=[hspec, hspec, pl.BlockSpec((blk, FF), lambda i: (i, 0))],
        out_shape=[
            jax.ShapeDtypeStruct((NP, FH), jnp.float32),
            jax.ShapeDtypeStruct((NP, FH), jnp.float32),
            jax.ShapeDtypeStruct((NP, FF), jnp.float32),
        ],
    )(xp, wl, wr, b.reshape(1, FF))


def _tc_comb_proj_body(sa_ref, sb_ref, pa_ref, pb_ref, q_ref, d_ref,
                       wl_ref, wr_ref, b_ref,
                       p2a_ref, p2b_ref, q2_ref, inv_ref):
    inv = 1.0 / (1.0 + d_ref[...][:, :1])
    inv_ref[...] = inv
    s = jnp.concatenate([sa_ref[...], sb_ref[...]], axis=1)
    p = jnp.concatenate([pa_ref[...], pb_ref[...]], axis=1)
    h = jnp.maximum((s + p) * inv + q_ref[...], 0.0)
    p2 = jnp.dot(h, wl_ref[...], preferred_element_type=jnp.float32)
    p2a_ref[...] = p2[:, :FH]
    p2b_ref[...] = p2[:, FH:]
    q2_ref[...] = (
        jnp.dot(h, wr_ref[...], preferred_element_type=jnp.float32)
        + b_ref[...]
    )


def _tc_comb_proj(sa, sb, pa, pb, q, deg, wl, wr, b):
    blk = 1024
    grid = NP // blk
    rspec = pl.BlockSpec((blk, FF), lambda i: (i, 0))
    hspec = pl.BlockSpec((blk, FH), lambda i: (i, 0))
    dspec = pl.BlockSpec((blk, 16), lambda i: (i, 0))
    wspec = pl.BlockSpec((FF, FF), lambda i: (0, 0))
    return pl.pallas_call(
        _tc_comb_proj_body,
        grid=(grid,),
        in_specs=[hspec, hspec, hspec, hspec, rspec, dspec, wspec, wspec,
                  pl.BlockSpec((1, FF), lambda i: (0, 0))],
        out_specs=[hspec, hspec, rspec,
                   pl.BlockSpec((blk, 1), lambda i: (i, 0))],
        out_shape=[
            jax.ShapeDtypeStruct((NP, FH), jnp.float32),
            jax.ShapeDtypeStruct((NP, FH), jnp.float32),
            jax.ShapeDtypeStruct((NP, FF), jnp.float32),
            jax.ShapeDtypeStruct((NP, 1), jnp.float32),
        ],
    )(sa, sb, pa, pb, q, deg, wl, wr, b.reshape(1, FF))


def _tc_comb_body(sa_ref, sb_ref, pa_ref, pb_ref, q_ref, inv_ref, o_ref):
    s = jnp.concatenate([sa_ref[...], sb_ref[...]], axis=1)
    p = jnp.concatenate([pa_ref[...], pb_ref[...]], axis=1)
    o_ref[...] = jnp.maximum((s + p) * inv_ref[...] + q_ref[...], 0.0)


def _tc_comb(sa, sb, pa, pb, q, inv):
    blk = 1024
    grid = NP // blk
    rspec = pl.BlockSpec((blk, FF), lambda i: (i, 0))
    hspec = pl.BlockSpec((blk, FH), lambda i: (i, 0))
    ispec = pl.BlockSpec((blk, 1), lambda i: (i, 0))
    return pl.pallas_call(
        _tc_comb_body,
        grid=(grid,),
        in_specs=[hspec, hspec, hspec, hspec, rspec, ispec],
        out_specs=rspec,
        out_shape=jax.ShapeDtypeStruct((NP, FF), jnp.float32),
    )(sa, sb, pa, pb, q, inv)


def _make_sc_scatter(compute_deg: bool):
    mesh = plsc.VectorSubcoreMesh(core_axis_name="c", subcore_axis_name="s")
    out_type = [jax.ShapeDtypeStruct((2, NP, FH), jnp.float32)]
    if compute_deg:
        out_type.append(jax.ShapeDtypeStruct((NP, 16), jnp.float32))
    scratch = []
    for _ in range(NBUF):
        scratch += [
            pltpu.VMEM((KSUB, 128), jnp.int32),
            pltpu.VMEM((KSUB, 128), jnp.int32),
            pltpu.VMEM((CHUNK, FH), jnp.float32),
            pltpu.SemaphoreType.DMA,
            pltpu.SemaphoreType.DMA,
        ]
    scratch += [
        pltpu.VMEM((16, FH), jnp.float32),
        pltpu.VMEM_SHARED((NP, FH), jnp.float32),
    ]
    if compute_deg:
        scratch += [
            pltpu.VMEM((128, 16), jnp.float32),
            pltpu.VMEM((64, 16), jnp.float32),
            pltpu.VMEM_SHARED((NP, 16), jnp.float32),
        ]

    def body(pa_hbm, pb_hbm, src_hbm, dst_hbm, *refs):
        if compute_deg:
            (sacc_hbm, deg_hbm, *bufrefs, zbuf, acc,
             onesb, zb16, dacc) = refs
        else:
            (sacc_hbm, *bufrefs, zbuf, acc) = refs
        bufs = [tuple(bufrefs[5 * k:5 * k + 5]) for k in range(NBUF)]
        c = lax.axis_index("c")
        s = lax.axis_index("s")

        z16 = jnp.zeros((16,), jnp.float32)
        for r in range(16):
            for k in range(FH // 16):
                zbuf[r, pl.ds(k * 16, 16)] = z16

        def zero_acc(i, _):
            pltpu.sync_copy(zbuf, acc.at[pl.ds(s * RPT + i * 16, 16)])
            return 0
        lax.fori_loop(0, RPT // 16, zero_acc, 0)

        if compute_deg:
            o16 = jnp.ones((16,), jnp.float32)

            def fill_ones(i, _):
                onesb[i, pl.ds(0, 16)] = o16
                return 0
            lax.fori_loop(0, 128, fill_ones, 0)

            def zero_zb16(i, _):
                zb16[i, pl.ds(0, 16)] = z16
                return 0
            lax.fori_loop(0, 64, zero_zb16, 0)

            @pl.when(c == 0)
            def _():
                def zero_dacc(i, _):
                    pltpu.sync_copy(
                        zb16, dacc.at[pl.ds(s * RPT + i * 64, 64)])
                    return 0
                lax.fori_loop(0, RPT // 64, zero_dacc, 0)

        plsc.subcore_barrier()

        def run(p_hbm, do_deg):
            def load_idx(buf, g):
                row0 = s * (EPT // 128) + g * KSUB
                pltpu.sync_copy(src_hbm.at[pl.ds(row0, KSUB)], buf[0])
                pltpu.sync_copy(dst_hbm.at[pl.ds(row0, KSUB)], buf[1])

            def issue_g(buf):
                for j in range(KSUB):
                    pltpu.async_copy(
                        p_hbm.at[buf[0].at[j]],
                        buf[2].at[pl.ds(j * 128, 128)], buf[3])

            def wait_g(buf):
                for j in range(KSUB):
                    pltpu.make_async_copy(
                        p_hbm.at[buf[0].at[j]],
                        buf[2].at[pl.ds(j * 128, 128)], buf[3]).wait()

            def issue_s(buf):
                for j in range(KSUB):
                    pltpu.async_copy(
                        buf[2].at[pl.ds(j * 128, 128)],
                        acc.at[buf[1].at[j]], buf[4], add=True)
                if do_deg:
                    for j in range(KSUB):
                        pltpu.async_copy(
                            onesb, dacc.at[buf[1].at[j]], buf[4], add=True)

            def wait_s(buf):
                for j in range(KSUB):
                    pltpu.make_async_copy(
                        buf[2].at[pl.ds(j * 128, 128)],
                        acc.at[buf[1].at[j]], buf[4]).wait()
                if do_deg:
                    for j in range(KSUB):
                        pltpu.make_async_copy(
                            onesb, dacc.at[buf[1].at[j]], buf[4]).wait()

            def chunk(cur, prev, cc, pf=None, wait_prev=True, guard=False):
                wait_g(cur)
                issue_s(cur)
                if wait_prev:
                    wait_s(prev)
                if pf is not None:
                    if guard:
                        @pl.when(pf <= NCHUNK - 1)
                        def _():
                            load_idx(prev, pf)
                            issue_g(prev)
                    else:
                        load_idx(prev, pf)
                        issue_g(prev)

            for k in range(NBUF - 1):
                load_idx(bufs[k], k)
                issue_g(bufs[k])
            chunk(bufs[0], bufs[NBUF - 1], 0, NBUF - 1, wait_prev=False)
            for r in range(1, NBUF):
                chunk(bufs[r], bufs[r - 1], r, r + NBUF - 1)

            def step(u, _):
                cbase = NBUF * u
                for r in range(NBUF):
                    chunk(bufs[r], bufs[r - 1], cbase + r,
                          cbase + r + NBUF - 1, guard=True)
                return 0
            lax.fori_loop(1, NCHUNK // NBUF, step, 0)

            wait_s(bufs[(NCHUNK - 1) % NBUF])

        @pl.when(c == 0)
        def _():
            run(pa_hbm, compute_deg)

        @pl.when(c == 1)
        def _():
            run(pb_hbm, False)

        plsc.subcore_barrier()

        pltpu.sync_copy(
            acc.at[pl.ds(s * RPT, RPT)],
            sacc_hbm.at[c, pl.ds(s * RPT, RPT)])

        if compute_deg:
            @pl.when(c == 0)
            def _():
                pltpu.sync_copy(
                    dacc.at[pl.ds(s * RPT, RPT)],
                    deg_hbm.at[pl.ds(s * RPT, RPT)])

    return pl.kernel(
        body, mesh=mesh, out_type=out_type, scratch_types=scratch,
        compiler_params=pltpu.CompilerParams(use_tc_tiling_on_sc=False))


_sc_scatter_deg = _make_sc_scatter(True)
_sc_scatter = _make_sc_scatter(False)


@jax.jit
def kernel(x, edge_index, W1l, W1r, b1, W2l, W2r, b2):
    xp = jnp.zeros((NP, FF), jnp.float32).at[:NN].set(x)
    pad = EP - EE
    srcp = jnp.concatenate(
        [edge_index[0], jnp.zeros((pad,), jnp.int32)]).reshape(EP // 128, 128)
    dstp = jnp.concatenate(
        [edge_index[1], jnp.full((pad,), NN, jnp.int32)]).reshape(EP // 128, 128)

    p1a, p1b, q1 = _tc_proj(xp, W1l, W1r, b1)
    sacc1, deg = _sc_scatter_deg(p1a, p1b, srcp, dstp)
    p2a, p2b, q2, inv = _tc_comb_proj(sacc1[0], sacc1[1], p1a, p1b, q1,
                                      deg, W2l, W2r, b2)
    (sacc2,) = _sc_scatter(p2a, p2b, srcp, dstp)
    out = _tc_comb(sacc2[0], sacc2[1], p2a, p2b, q2, inv)
    return out[:NN]


# speedup vs baseline: 7.8081x; 1.0379x over previous
import jax
import jax.numpy as jnp
from jax import lax
from jax.experimental import pallas as pl
from jax.experimental.pallas import tpu as pltpu
from jax.experimental.pallas import tpu_sc as plsc

NN = 10000
NP = 10240
FF = 128
FH = 64
EE = 320000
EP = 327680
EPT = EP // 16
CHUNK = 128
NBUF = 4
NCHUNK = EPT // CHUNK
RPT = NP // 16


def _tc_proj_body(x_ref, wl_ref, wr_ref, b_ref, pa_ref, pb_ref, q_ref):
    xb = x_ref[...]
    p = jnp.dot(xb, wl_ref[...], preferred_element_type=jnp.float32)
    pa_ref[...] = p[:, :FH]
    pb_ref[...] = p[:, FH:]
    q_ref[...] = (
        jnp.dot(xb, wr_ref[...], preferred_element_type=jnp.float32)
        + b_ref[...]
    )


def _tc_proj(xp, wl, wr, b):
    blk = 1024
    grid = NP // blk
    hspec = pl.BlockSpec((blk, FH), lambda i: (i, 0))
    return pl.pallas_call(
        _tc_proj_body,
        grid=(grid,),
        in_specs=[
            pl.BlockSpec((blk, FF), lambda i: (i, 0)),
            pl.BlockSpec((FF, FF), lambda i: (0, 0)),
            pl.BlockSpec((FF, FF), lambda i: (0, 0)),
            pl.BlockSpec((1, FF), lambda i: (0, 0)),
        ],
        out_specs=[hspec, hspec, pl.BlockSpec((blk, FF), lambda i: (i, 0))],
        out_shape=[
            jax.ShapeDtypeStruct((NP, FH), jnp.float32),
            jax.ShapeDtypeStruct((NP, FH), jnp.float32),
            jax.ShapeDtypeStruct((NP, FF), jnp.float32),
        ],
    )(xp, wl, wr, b.reshape(1, FF))


def _tc_comb_proj_body(sa_ref, sb_ref, pa_ref, pb_ref, q_ref, d_ref,
                       wl_ref, wr_ref, b_ref,
                       p2a_ref, p2b_ref, q2_ref, inv_ref):
    inv = 1.0 / (1.0 + d_ref[...][:, :1])
    inv_ref[...] = inv
    s = jnp.concatenate([sa_ref[...], sb_ref[...]], axis=1)
    p = jnp.concatenate([pa_ref[...], pb_ref[...]], axis=1)
    h = jnp.maximum((s + p) * inv + q_ref[...], 0.0)
    p2 = jnp.dot(h, wl_ref[...], preferred_element_type=jnp.float32)
    p2a_ref[...] = p2[:, :FH]
    p2b_ref[...] = p2[:, FH:]
    q2_ref[...] = (
        jnp.dot(h, wr_ref[...], preferred_element_type=jnp.float32)
        + b_ref[...]
    )


def _tc_comb_proj(sa, sb, pa, pb, q, deg, wl, wr, b):
    blk = 1024
    grid = NP // blk
    rspec = pl.BlockSpec((blk, FF), lambda i: (i, 0))
    hspec = pl.BlockSpec((blk, FH), lambda i: (i, 0))
    dspec = pl.BlockSpec((blk, 16), lambda i: (i, 0))
    wspec = pl.BlockSpec((FF, FF), lambda i: (0, 0))
    return pl.pallas_call(
        _tc_comb_proj_body,
        grid=(grid,),
        in_specs=[hspec, hspec, hspec, hspec, rspec, dspec, wspec, wspec,
                  pl.BlockSpec((1, FF), lambda i: (0, 0))],
        out_specs=[hspec, hspec, rspec,
                   pl.BlockSpec((blk, 1), lambda i: (i, 0))],
        out_shape=[
            jax.ShapeDtypeStruct((NP, FH), jnp.float32),
            jax.ShapeDtypeStruct((NP, FH), jnp.float32),
            jax.ShapeDtypeStruct((NP, FF), jnp.float32),
            jax.ShapeDtypeStruct((NP, 1), jnp.float32),
        ],
    )(sa, sb, pa, pb, q, deg, wl, wr, b.reshape(1, FF))


def _tc_comb_body(sa_ref, sb_ref, pa_ref, pb_ref, q_ref, inv_ref, o_ref):
    s = jnp.concatenate([sa_ref[...], sb_ref[...]], axis=1)
    p = jnp.concatenate([pa_ref[...], pb_ref[...]], axis=1)
    o_ref[...] = jnp.maximum((s + p) * inv_ref[...] + q_ref[...], 0.0)


def _tc_comb(sa, sb, pa, pb, q, inv):
    blk = 1024
    grid = NP // blk
    rspec = pl.BlockSpec((blk, FF), lambda i: (i, 0))
    hspec = pl.BlockSpec((blk, FH), lambda i: (i, 0))
    ispec = pl.BlockSpec((blk, 1), lambda i: (i, 0))
    return pl.pallas_call(
        _tc_comb_body,
        grid=(grid,),
        in_specs=[hspec, hspec, hspec, hspec, rspec, ispec],
        out_specs=rspec,
        out_shape=jax.ShapeDtypeStruct((NP, FF), jnp.float32),
    )(sa, sb, pa, pb, q, inv)


def _make_sc_scatter(compute_deg: bool):
    mesh = plsc.VectorSubcoreMesh(core_axis_name="c", subcore_axis_name="s")
    out_type = [jax.ShapeDtypeStruct((2, NP, FH), jnp.float32)]
    if compute_deg:
        out_type.append(jax.ShapeDtypeStruct((NP, 16), jnp.float32))
    scratch = []
    for _ in range(NBUF):
        scratch += [
            pltpu.VMEM((CHUNK, FH), jnp.float32),
            pltpu.SemaphoreType.DMA,
            pltpu.SemaphoreType.DMA,
        ]
    scratch += [
        pltpu.VMEM((NCHUNK, 128), jnp.int32),
        pltpu.VMEM((NCHUNK, 128), jnp.int32),
        pltpu.SemaphoreType.DMA,
        pltpu.VMEM((16, FH), jnp.float32),
        pltpu.VMEM_SHARED((NP, FH), jnp.float32),
    ]
    if compute_deg:
        scratch += [
            pltpu.VMEM((128, 16), jnp.float32),
            pltpu.VMEM((64, 16), jnp.float32),
            pltpu.VMEM_SHARED((NP, 16), jnp.float32),
        ]

    def body(pa_hbm, pb_hbm, src_hbm, dst_hbm, *refs):
        if compute_deg:
            (sacc_hbm, deg_hbm, *bufrefs, srcpre, dstpre, semi, zbuf, acc,
             onesb, zb16, dacc) = refs
        else:
            (sacc_hbm, *bufrefs, srcpre, dstpre, semi, zbuf, acc) = refs
        bufs = [tuple(bufrefs[3 * k:3 * k + 3]) for k in range(NBUF)]
        c = lax.axis_index("c")
        s = lax.axis_index("s")

        pltpu.async_copy(
            src_hbm.at[pl.ds(s * NCHUNK, NCHUNK)], srcpre, semi)
        pltpu.async_copy(
            dst_hbm.at[pl.ds(s * NCHUNK, NCHUNK)], dstpre, semi)

        z16 = jnp.zeros((16,), jnp.float32)
        for r in range(16):
            for k in range(FH // 16):
                zbuf[r, pl.ds(k * 16, 16)] = z16

        def zero_acc(i, _):
            pltpu.sync_copy(zbuf, acc.at[pl.ds(s * RPT + i * 16, 16)])
            return 0
        lax.fori_loop(0, RPT // 16, zero_acc, 0)

        if compute_deg:
            o16 = jnp.ones((16,), jnp.float32)

            def fill_ones(i, _):
                onesb[i, pl.ds(0, 16)] = o16
                return 0
            lax.fori_loop(0, 128, fill_ones, 0)

            def zero_zb16(i, _):
                zb16[i, pl.ds(0, 16)] = z16
                return 0
            lax.fori_loop(0, 64, zero_zb16, 0)

            @pl.when(c == 0)
            def _():
                def zero_dacc(i, _):
                    pltpu.sync_copy(
                        zb16, dacc.at[pl.ds(s * RPT + i * 64, 64)])
                    return 0
                lax.fori_loop(0, RPT // 64, zero_dacc, 0)

        plsc.subcore_barrier()

        pltpu.make_async_copy(
            src_hbm.at[pl.ds(s * NCHUNK, NCHUNK)], srcpre, semi).wait()
        pltpu.make_async_copy(
            dst_hbm.at[pl.ds(s * NCHUNK, NCHUNK)], dstpre, semi).wait()

        def run(p_hbm, do_deg):
            def issue_g(buf, g):
                pltpu.async_copy(p_hbm.at[srcpre.at[g]], buf[0], buf[1])

            def wait_g(buf, g):
                pltpu.make_async_copy(
                    p_hbm.at[srcpre.at[g]], buf[0], buf[1]).wait()

            def issue_s(buf, g):
                pltpu.async_copy(
                    buf[0], acc.at[dstpre.at[g]], buf[2], add=True)
                if do_deg:
                    pltpu.async_copy(
                        onesb, dacc.at[dstpre.at[g]], buf[2], add=True)

            def wait_s(buf, g):
                pltpu.make_async_copy(
                    buf[0], acc.at[dstpre.at[g]], buf[2]).wait()
                if do_deg:
                    pltpu.make_async_copy(
                        onesb, dacc.at[dstpre.at[g]], buf[2]).wait()

            def chunk(cur, prev, cc, pf=None, wait_prev=True, guard=False):
                wait_g(cur, cc)
                issue_s(cur, cc)
                if wait_prev:
                    wait_s(prev, cc - 1)
                if pf is not None:
                    if guard:
                        @pl.when(pf <= NCHUNK - 1)
                        def _():
                            issue_g(prev, pf)
                    else:
                        issue_g(prev, pf)

            for k in range(NBUF - 1):
                issue_g(bufs[k], k)
            chunk(bufs[0], bufs[NBUF - 1], 0, NBUF - 1, wait_prev=False)
            for r in range(1, NBUF):
                chunk(bufs[r], bufs[r - 1], r, r + NBUF - 1)

            def step(u, _):
                cbase = NBUF * u
                for r in range(NBUF):
                    chunk(bufs[r], bufs[r - 1], cbase + r,
                          cbase + r + NBUF - 1, guard=True)
                return 0
            lax.fori_loop(1, NCHUNK // NBUF, step, 0)

            wait_s(bufs[(NCHUNK - 1) % NBUF], NCHUNK - 1)

        @pl.when(c == 0)
        def _():
            run(pa_hbm, compute_deg)

        @pl.when(c == 1)
        def _():
            run(pb_hbm, False)

        plsc.subcore_barrier()

        pltpu.sync_copy(
            acc.at[pl.ds(s * RPT, RPT)],
            sacc_hbm.at[c, pl.ds(s * RPT, RPT)])

        if compute_deg:
            @pl.when(c == 0)
            def _():
                pltpu.sync_copy(
                    dacc.at[pl.ds(s * RPT, RPT)],
                    deg_hbm.at[pl.ds(s * RPT, RPT)])

    return pl.kernel(
        body, mesh=mesh, out_type=out_type, scratch_types=scratch,
        compiler_params=pltpu.CompilerParams(use_tc_tiling_on_sc=False))


_sc_scatter_deg = _make_sc_scatter(True)
_sc_scatter = _make_sc_scatter(False)


@jax.jit
def kernel(x, edge_index, W1l, W1r, b1, W2l, W2r, b2):
    xp = jnp.zeros((NP, FF), jnp.float32).at[:NN].set(x)
    pad = EP - EE
    srcp = jnp.concatenate(
        [edge_index[0], jnp.zeros((pad,), jnp.int32)]).reshape(EP // 128, 128)
    dstp = jnp.concatenate(
        [edge_index[1], jnp.full((pad,), NN, jnp.int32)]).reshape(EP // 128, 128)

    p1a, p1b, q1 = _tc_proj(xp, W1l, W1r, b1)
    sacc1, deg = _sc_scatter_deg(p1a, p1b, srcp, dstp)
    p2a, p2b, q2, inv = _tc_comb_proj(sacc1[0], sacc1[1], p1a, p1b, q1,
                                      deg, W2l, W2r, b2)
    (sacc2,) = _sc_scatter(p2a, p2b, srcp, dstp)
    out = _tc_comb(sacc2[0], sacc2[1], p2a, p2b, q2, inv)
    return out[:NN]


# speedup vs baseline: 7.8115x; 1.0004x over previous
import jax
import jax.numpy as jnp
from jax import lax
from jax.experimental import pallas as pl
from jax.experimental.pallas import tpu as pltpu
from jax.experimental.pallas import tpu_sc as plsc

NN = 10000
NP = 10240
FF = 128
FH = 64
EE = 320000
EP = 327680
EPT = EP // 16
CHUNK = 128
NBUF = 4
NCHUNK = EPT // CHUNK
RPT = NP // 16


def _tc_proj_body(x_ref, wl_ref, wr_ref, b_ref, pa_ref, pb_ref, q_ref):
    xb = x_ref[...]
    p = jnp.dot(xb, wl_ref[...], preferred_element_type=jnp.float32)
    pa_ref[...] = p[:, :FH]
    pb_ref[...] = p[:, FH:]
    q_ref[...] = (
        jnp.dot(xb, wr_ref[...], preferred_element_type=jnp.float32)
        + b_ref[...]
    )


def _tc_proj(xp, wl, wr, b):
    blk = 1024
    grid = NP // blk
    hspec = pl.BlockSpec((blk, FH), lambda i: (i, 0))
    return pl.pallas_call(
        _tc_proj_body,
        grid=(grid,),
        in_specs=[
            pl.BlockSpec((blk, FF), lambda i: (i, 0)),
            pl.BlockSpec((FF, FF), lambda i: (0, 0)),
            pl.BlockSpec((FF, FF), lambda i: (0, 0)),
            pl.BlockSpec((1, FF), lambda i: (0, 0)),
        ],
        out_specs=[hspec, hspec, pl.BlockSpec((blk, FF), lambda i: (i, 0))],
        out_shape=[
            jax.ShapeDtypeStruct((NP, FH), jnp.float32),
            jax.ShapeDtypeStruct((NP, FH), jnp.float32),
            jax.ShapeDtypeStruct((NP, FF), jnp.float32),
        ],
    )(xp, wl, wr, b.reshape(1, FF))


def _tc_comb_proj_body(sa_ref, sb_ref, pa_ref, pb_ref, q_ref, d_ref,
                       wl_ref, wr_ref, b_ref,
                       p2a_ref, p2b_ref, q2_ref, inv_ref):
    inv = 1.0 / (1.0 + d_ref[...][:, :1])
    inv_ref[...] = inv
    s = jnp.concatenate([sa_ref[...], sb_ref[...]], axis=1)
    p = jnp.concatenate([pa_ref[...], pb_ref[...]], axis=1)
    h = jnp.maximum((s + p) * inv + q_ref[...], 0.0)
    p2 = jnp.dot(h, wl_ref[...], preferred_element_type=jnp.float32)
    p2a_ref[...] = p2[:, :FH]
    p2b_ref[...] = p2[:, FH:]
    q2_ref[...] = (
        jnp.dot(h, wr_ref[...], preferred_element_type=jnp.float32)
        + b_ref[...]
    )


def _tc_comb_proj(sa, sb, pa, pb, q, deg, wl, wr, b):
    blk = 1024
    grid = NP // blk
    rspec = pl.BlockSpec((blk, FF), lambda i: (i, 0))
    hspec = pl.BlockSpec((blk, FH), lambda i: (i, 0))
    dspec = pl.BlockSpec((blk, 16), lambda i: (i, 0))
    wspec = pl.BlockSpec((FF, FF), lambda i: (0, 0))
    return pl.pallas_call(
        _tc_comb_proj_body,
        grid=(grid,),
        in_specs=[hspec, hspec, hspec, hspec, rspec, dspec, wspec, wspec,
                  pl.BlockSpec((1, FF), lambda i: (0, 0))],
        out_specs=[hspec, hspec, rspec,
                   pl.BlockSpec((blk, 1), lambda i: (i, 0))],
        out_shape=[
            jax.ShapeDtypeStruct((NP, FH), jnp.float32),
            jax.ShapeDtypeStruct((NP, FH), jnp.float32),
            jax.ShapeDtypeStruct((NP, FF), jnp.float32),
            jax.ShapeDtypeStruct((NP, 1), jnp.float32),
        ],
    )(sa, sb, pa, pb, q, deg, wl, wr, b.reshape(1, FF))


def _tc_comb_body(sa_ref, sb_ref, pa_ref, pb_ref, q_ref, inv_ref, o_ref):
    s = jnp.concatenate([sa_ref[...], sb_ref[...]], axis=1)
    p = jnp.concatenate([pa_ref[...], pb_ref[...]], axis=1)
    o_ref[...] = jnp.maximum((s + p) * inv_ref[...] + q_ref[...], 0.0)


def _tc_comb(sa, sb, pa, pb, q, inv):
    blk = 1024
    grid = NP // blk
    rspec = pl.BlockSpec((blk, FF), lambda i: (i, 0))
    hspec = pl.BlockSpec((blk, FH), lambda i: (i, 0))
    ispec = pl.BlockSpec((blk, 1), lambda i: (i, 0))
    return pl.pallas_call(
        _tc_comb_body,
        grid=(grid,),
        in_specs=[hspec, hspec, hspec, hspec, rspec, ispec],
        out_specs=rspec,
        out_shape=jax.ShapeDtypeStruct((NP, FF), jnp.float32),
    )(sa, sb, pa, pb, q, inv)


def _make_sc_scatter(compute_deg: bool, NBUF: int = NBUF):
    mesh = plsc.VectorSubcoreMesh(core_axis_name="c", subcore_axis_name="s")
    out_type = [jax.ShapeDtypeStruct((2, NP, FH), jnp.float32)]
    if compute_deg:
        out_type.append(jax.ShapeDtypeStruct((NP, 16), jnp.float32))
    scratch = []
    for _ in range(NBUF):
        scratch += [
            pltpu.VMEM((CHUNK, FH), jnp.float32),
            pltpu.SemaphoreType.DMA,
            pltpu.SemaphoreType.DMA,
        ]
    scratch += [
        pltpu.VMEM((NCHUNK, 128), jnp.int32),
        pltpu.VMEM((NCHUNK, 128), jnp.int32),
        pltpu.SemaphoreType.DMA,
        pltpu.VMEM((16, FH), jnp.float32),
        pltpu.VMEM_SHARED((NP, FH), jnp.float32),
    ]
    if compute_deg:
        scratch += [
            pltpu.VMEM((128, 16), jnp.float32),
            pltpu.VMEM((64, 16), jnp.float32),
            pltpu.VMEM_SHARED((NP, 16), jnp.float32),
        ]

    def body(pa_hbm, pb_hbm, src_hbm, dst_hbm, *refs):
        if compute_deg:
            (sacc_hbm, deg_hbm, *bufrefs, srcpre, dstpre, semi, zbuf, acc,
             onesb, zb16, dacc) = refs
        else:
            (sacc_hbm, *bufrefs, srcpre, dstpre, semi, zbuf, acc) = refs
        bufs = [tuple(bufrefs[3 * k:3 * k + 3]) for k in range(NBUF)]
        c = lax.axis_index("c")
        s = lax.axis_index("s")

        pltpu.async_copy(
            src_hbm.at[pl.ds(s * NCHUNK, NCHUNK)], srcpre, semi)
        pltpu.async_copy(
            dst_hbm.at[pl.ds(s * NCHUNK, NCHUNK)], dstpre, semi)

        z16 = jnp.zeros((16,), jnp.float32)
        for r in range(16):
            for k in range(FH // 16):
                zbuf[r, pl.ds(k * 16, 16)] = z16

        def zero_acc(i, _):
            pltpu.sync_copy(zbuf, acc.at[pl.ds(s * RPT + i * 16, 16)])
            return 0
        lax.fori_loop(0, RPT // 16, zero_acc, 0)

        if compute_deg:
            o16 = jnp.ones((16,), jnp.float32)

            def fill_ones(i, _):
                onesb[i, pl.ds(0, 16)] = o16
                return 0
            lax.fori_loop(0, 128, fill_ones, 0)

            def zero_zb16(i, _):
                zb16[i, pl.ds(0, 16)] = z16
                return 0
            lax.fori_loop(0, 64, zero_zb16, 0)

            @pl.when(c == 0)
            def _():
                def zero_dacc(i, _):
                    pltpu.sync_copy(
                        zb16, dacc.at[pl.ds(s * RPT + i * 64, 64)])
                    return 0
                lax.fori_loop(0, RPT // 64, zero_dacc, 0)

        plsc.subcore_barrier()

        pltpu.make_async_copy(
            src_hbm.at[pl.ds(s * NCHUNK, NCHUNK)], srcpre, semi).wait()
        pltpu.make_async_copy(
            dst_hbm.at[pl.ds(s * NCHUNK, NCHUNK)], dstpre, semi).wait()

        def run(p_hbm, do_deg):
            def issue_g(buf, g):
                pltpu.async_copy(p_hbm.at[srcpre.at[g]], buf[0], buf[1])

            def wait_g(buf, g):
                pltpu.make_async_copy(
                    p_hbm.at[srcpre.at[g]], buf[0], buf[1]).wait()

            def issue_s(buf, g):
                pltpu.async_copy(
                    buf[0], acc.at[dstpre.at[g]], buf[2], add=True)
                if do_deg:
                    pltpu.async_copy(
                        onesb, dacc.at[dstpre.at[g]], buf[2], add=True)

            def wait_s(buf, g):
                pltpu.make_async_copy(
                    buf[0], acc.at[dstpre.at[g]], buf[2]).wait()
                if do_deg:
                    pltpu.make_async_copy(
                        onesb, dacc.at[dstpre.at[g]], buf[2]).wait()

            def chunk(cur, prev, cc, pf=None, wait_prev=True, guard=False):
                wait_g(cur, cc)
                issue_s(cur, cc)
                if wait_prev:
                    wait_s(prev, cc - 1)
                if pf is not None:
                    if guard:
                        @pl.when(pf <= NCHUNK - 1)
                        def _():
                            issue_g(prev, pf)
                    else:
                        issue_g(prev, pf)

            for k in range(NBUF - 1):
                issue_g(bufs[k], k)
            chunk(bufs[0], bufs[NBUF - 1], 0, NBUF - 1, wait_prev=False)
            for r in range(1, NBUF):
                chunk(bufs[r], bufs[r - 1], r, r + NBUF - 1)

            def step(u, _):
                cbase = NBUF * u
                for r in range(NBUF):
                    chunk(bufs[r], bufs[r - 1], cbase + r,
                          cbase + r + NBUF - 1, guard=True)
                return 0
            lax.fori_loop(1, NCHUNK // NBUF, step, 0)

            wait_s(bufs[(NCHUNK - 1) % NBUF], NCHUNK - 1)

        @pl.when(c == 0)
        def _():
            run(pa_hbm, compute_deg)

        @pl.when(c == 1)
        def _():
            run(pb_hbm, False)

        plsc.subcore_barrier()

        pltpu.sync_copy(
            acc.at[pl.ds(s * RPT, RPT)],
            sacc_hbm.at[c, pl.ds(s * RPT, RPT)])

        if compute_deg:
            @pl.when(c == 0)
            def _():
                pltpu.sync_copy(
                    dacc.at[pl.ds(s * RPT, RPT)],
                    deg_hbm.at[pl.ds(s * RPT, RPT)])

    return pl.kernel(
        body, mesh=mesh, out_type=out_type, scratch_types=scratch,
        compiler_params=pltpu.CompilerParams(use_tc_tiling_on_sc=False))


_sc_scatter_deg = _make_sc_scatter(True, 4)
_sc_scatter = _make_sc_scatter(False, 5)


@jax.jit
def kernel(x, edge_index, W1l, W1r, b1, W2l, W2r, b2):
    xp = jnp.zeros((NP, FF), jnp.float32).at[:NN].set(x)
    pad = EP - EE
    srcp = jnp.concatenate(
        [edge_index[0], jnp.zeros((pad,), jnp.int32)]).reshape(EP // 128, 128)
    dstp = jnp.concatenate(
        [edge_index[1], jnp.full((pad,), NN, jnp.int32)]).reshape(EP // 128, 128)

    p1a, p1b, q1 = _tc_proj(xp, W1l, W1r, b1)
    sacc1, deg = _sc_scatter_deg(p1a, p1b, srcp, dstp)
    p2a, p2b, q2, inv = _tc_comb_proj(sacc1[0], sacc1[1], p1a, p1b, q1,
                                      deg, W2l, W2r, b2)
    (sacc2,) = _sc_scatter(p2a, p2b, srcp, dstp)
    out = _tc_comb(sacc2[0], sacc2[1], p2a, p2b, q2, inv)
    return out[:NN]
